# Initial kernel scaffold; baseline (speedup 1.0000x reference)
import jax
import jax.numpy as jnp
from jax.experimental import pallas as pl


def kernel(x, edge_index, batch, Ws1, bs1, Ws2, bs2, Wg, bg):
    raise NotImplementedError("write your pallas kernel here")


# speedup vs baseline: 7.1333x; 7.1333x over previous
import functools

import jax
import jax.numpy as jnp
from jax import lax
from jax.experimental import pallas as pl
from jax.experimental.pallas import tpu as pltpu
from jax.experimental.pallas import tpu_sc as plsc

G = 256
_NC, _NS = 2, 16
_NW = _NC * _NS
_CH = 128


def _sc_agg_body(K, ZCH, src_hbm, dst_hbm, h_hbm, zeros_hbm, out_hbm,
                 src_v, dst_v, rows_v, agg_sp, gsem):
    cid = lax.axis_index("c")
    sid = lax.axis_index("s")
    pltpu.sync_copy(zeros_hbm, agg_sp.at[pl.ds(sid * ZCH, ZCH)])
    plsc.subcore_barrier()
    w = sid * _NC + cid
    pltpu.sync_copy(src_hbm.at[w], src_v)
    pltpu.sync_copy(dst_hbm.at[w], dst_v)
    for j in range(K):
        pltpu.async_copy(h_hbm.at[src_v.at[j]], rows_v, gsem).wait()
        pltpu.sync_copy(rows_v, agg_sp.at[dst_v.at[j]], add=True)
    plsc.subcore_barrier()
    pltpu.sync_copy(agg_sp.at[pl.ds(sid * ZCH, ZCH)],
                    out_hbm.at[cid, pl.ds(sid * ZCH, ZCH)])


def _make_sc_agg(N, D, K, NSP):
    mesh = plsc.VectorSubcoreMesh(core_axis_name="c", subcore_axis_name="s")
    body = functools.partial(_sc_agg_body, K, NSP // _NS)
    return pl.kernel(
        body,
        out_type=jax.ShapeDtypeStruct((_NC, NSP, D), jnp.float32),
        mesh=mesh,
        scratch_types=[
            pltpu.VMEM((K, _CH), jnp.int32),
            pltpu.VMEM((K, _CH), jnp.int32),
            pltpu.VMEM((_CH, D), jnp.float32),
            pltpu.VMEM_SHARED((NSP, D), jnp.float32),
            pltpu.SemaphoreType.DMA,
        ],
    )


def _mlp_body(h_ref, a0_ref, a1_ref, w1_ref, b1_ref, w2_ref, b2_ref, o_ref):
    z = h_ref[...] + a0_ref[...] + a1_ref[...]
    t = jnp.dot(z, w1_ref[...], preferred_element_type=jnp.float32) + b1_ref[...]
    t = jnp.where(t > 0.0, t, jnp.exp(t) - 1.0)
    o_ref[...] = (jnp.dot(t, w2_ref[...], preferred_element_type=jnp.float32)
                  + b2_ref[...])


def _mlp(h, a0, a1, W1, b1, W2, b2, BLK):
    N, D = h.shape
    H = W2.shape[1]
    nb = N // BLK
    row = lambda i: (i, 0)
    full = lambda i: (0, 0)
    return pl.pallas_call(
        _mlp_body,
        grid=(nb,),
        in_specs=[
            pl.BlockSpec((BLK, D), row),
            pl.BlockSpec((BLK, D), row),
            pl.BlockSpec((BLK, D), row),
            pl.BlockSpec((D, H), full),
            pl.BlockSpec((1, H), full),
            pl.BlockSpec((H, H), full),
            pl.BlockSpec((1, H), full),
        ],
        out_specs=pl.BlockSpec((BLK, H), row),
        out_shape=jax.ShapeDtypeStruct((N, H), jnp.float32),
    )(h, a0, a1, W1, b1.reshape(1, H), W2, b2.reshape(1, H))


def _readout_body(nb, BLK, h_ref, brow_ref, bcol_ref, wg_ref, bg_ref,
                  s_ref, m_ref, cnt_ref):
    i = pl.program_id(0)
    H = h_ref.shape[1]
    h = h_ref[...]
    b_row = brow_ref[0]
    b_col = bcol_ref[...]
    w = jax.nn.sigmoid(jnp.dot(h, wg_ref[...],
                               preferred_element_type=jnp.float32)
                       + bg_ref[...])
    wh = w * h
    giT = lax.broadcasted_iota(jnp.int32, (G, BLK), 0)
    onehotT = (giT == b_row).astype(jnp.float32)
    s_c = jnp.dot(onehotT, wh, preferred_element_type=jnp.float32)
    ones = jnp.ones((BLK, H), jnp.float32)
    cnt_c = jnp.dot(onehotT, ones, preferred_element_type=jnp.float32)
    v = h
    s = 1
    while s < BLK:
        vs = jnp.concatenate([v[:s], v[:-s]], axis=0)
        bs_ = jnp.concatenate([jnp.full((s, 1), -1, jnp.int32), b_col[:-s]],
                              axis=0)
        same = b_col == bs_
        v = jnp.where(same, jnp.maximum(v, vs), v)
        s *= 2
    nbrow = jnp.concatenate([b_row[:, 1:], jnp.full((1, 1), -1, jnp.int32)],
                            axis=1)
    is_last = (b_row != nbrow).astype(jnp.float32)
    selT = onehotT * is_last
    m_c = jnp.dot(selT, v, preferred_element_type=jnp.float32)
    m_c = jnp.where(cnt_c > 0.0, m_c, -3e38)

    @pl.when(i == 0)
    def _init():
        s_ref[...] = s_c
        m_ref[...] = m_c
        cnt_ref[...] = cnt_c

    @pl.when(i > 0)
    def _acc():
        s_ref[...] += s_c
        m_ref[...] = jnp.maximum(m_ref[...], m_c)
        cnt_ref[...] += cnt_c

    @pl.when(i == nb - 1)
    def _fin():
        m_ref[...] = jnp.where(cnt_ref[...] > 0.0, m_ref[...], 0.0)


def _readout(h, batch, Wg, bg, BLK):
    N, H = h.shape
    nb = N // BLK
    brow = batch.reshape(nb, 1, BLK)
    bcol = batch.reshape(N, 1)
    full = lambda i: (0, 0)
    s, m = pl.pallas_call(
        functools.partial(_readout_body, nb, BLK),
        grid=(nb,),
        in_specs=[
            pl.BlockSpec((BLK, H), lambda i: (i, 0)),
            pl.BlockSpec((1, 1, BLK), lambda i: (i, 0, 0)),
            pl.BlockSpec((BLK, 1), lambda i: (i, 0)),
            pl.BlockSpec((H, 1), full),
            pl.BlockSpec((1, 1), full),
        ],
        out_specs=[pl.BlockSpec((G, H), full), pl.BlockSpec((G, H), full)],
        out_shape=[jax.ShapeDtypeStruct((G, H), jnp.float32),
                   jax.ShapeDtypeStruct((G, H), jnp.float32)],
        scratch_shapes=[pltpu.VMEM((G, H), jnp.float32)],
        compiler_params=pltpu.CompilerParams(
            dimension_semantics=("arbitrary",)),
    )(h, brow, bcol, Wg, bg.reshape(1, 1))
    return s, m


def kernel(x, edge_index, batch, Ws1, bs1, Ws2, bs2, Wg, bg):
    N, D = x.shape
    L, _, H = Ws1.shape
    E = edge_index.shape[1]
    BLK = 1000

    K = -(-E // (_NW * _CH))
    EP = _NW * K * _CH
    NSP = _NS * (-(-(N + 1) // (_NS * 8)) * 8)
    NDUM = NSP - N

    pad = EP - E
    apad = jnp.arange(pad, dtype=jnp.int32)
    src3 = jnp.concatenate([edge_index[0], apad % N]).reshape(_NW, K, _CH)
    dst3 = jnp.concatenate([edge_index[1], N + apad % NDUM]).reshape(_NW, K, _CH)
    zeros = jnp.zeros((NSP // _NS, D), jnp.float32)

    sc_agg = _make_sc_agg(N, D, K, NSP)

    h = x
    for l in range(L):
        agg2 = sc_agg(src3, dst3, h, zeros)
        h = _mlp(h, agg2[0, :N], agg2[1, :N], Ws1[l], bs1[l], Ws2[l], bs2[l],
                 BLK)
    s, m = _readout(h, batch, Wg, bg, BLK)
    return jnp.concatenate([s, m], axis=1)


# speedup vs baseline: 10.4268x; 1.4617x over previous
import functools

import jax
import jax.numpy as jnp
from jax import lax
from jax.experimental import pallas as pl
from jax.experimental.pallas import tpu as pltpu
from jax.experimental.pallas import tpu_sc as plsc

G = 256
_NC, _NS = 2, 16
_NW = _NC * _NS
_CH = 128


_JW = 8


def _sc_agg_body(K, ZCH, src_hbm, dst_hbm, h_hbm, zeros_hbm, out_hbm,
                 srcw0, srcw1, dstw0, dstw1, rows0, rows1, agg_sp,
                 isem0, isem1, gsem0, gsem1):
    cid = lax.axis_index("c")
    sid = lax.axis_index("s")
    pltpu.sync_copy(zeros_hbm, agg_sp.at[pl.ds(sid * ZCH, ZCH)])
    plsc.subcore_barrier()
    w = sid * _NC + cid
    srcw = (srcw0, srcw1)
    dstw = (dstw0, dstw1)
    isem = (isem0, isem1)
    rows = (rows0, rows1)
    gsem = (gsem0, gsem1)
    nwin = K // _JW
    pltpu.sync_copy(src_hbm.at[w, pl.ds(0, _JW)], srcw0)
    pltpu.sync_copy(dst_hbm.at[w, pl.ds(0, _JW)], dstw0)
    gdesc = [None, None]
    idesc = [[None, None], [None, None]]
    gdesc[0] = pltpu.async_copy(h_hbm.at[srcw0.at[0]], rows0, gsem0)
    for win in range(nwin):
        wb = win & 1
        if win + 1 < nwin:
            nb_ = 1 - wb
            idesc[nb_][0] = pltpu.async_copy(
                src_hbm.at[w, pl.ds((win + 1) * _JW, _JW)], srcw[nb_],
                isem[nb_])
            idesc[nb_][1] = pltpu.async_copy(
                dst_hbm.at[w, pl.ds((win + 1) * _JW, _JW)], dstw[nb_],
                isem[nb_])
        for j2 in range(_JW):
            j = win * _JW + j2
            b = j & 1
            if j + 1 < K:
                if j2 + 1 < _JW:
                    nsrc = srcw[wb].at[j2 + 1]
                else:
                    idesc[1 - wb][0].wait()
                    idesc[1 - wb][1].wait()
                    nsrc = srcw[1 - wb].at[0]
                gdesc[1 - b] = pltpu.async_copy(h_hbm.at[nsrc], rows[1 - b],
                                                gsem[1 - b])
            gdesc[b].wait()
            pltpu.sync_copy(rows[b], agg_sp.at[dstw[wb].at[j2]], add=True)
    plsc.subcore_barrier()
    pltpu.sync_copy(agg_sp.at[pl.ds(sid * ZCH, ZCH)],
                    out_hbm.at[cid, pl.ds(sid * ZCH, ZCH)])


def _make_sc_agg(N, D, K, NSP):
    mesh = plsc.VectorSubcoreMesh(core_axis_name="c", subcore_axis_name="s")
    body = functools.partial(_sc_agg_body, K, NSP // _NS)
    return pl.kernel(
        body,
        out_type=jax.ShapeDtypeStruct((_NC, NSP, D), jnp.float32),
        mesh=mesh,
        scratch_types=[
            pltpu.VMEM((_JW, _CH), jnp.int32),
            pltpu.VMEM((_JW, _CH), jnp.int32),
            pltpu.VMEM((_JW, _CH), jnp.int32),
            pltpu.VMEM((_JW, _CH), jnp.int32),
            pltpu.VMEM((_CH, D), jnp.float32),
            pltpu.VMEM((_CH, D), jnp.float32),
            pltpu.VMEM_SHARED((NSP, D), jnp.float32),
            pltpu.SemaphoreType.DMA,
            pltpu.SemaphoreType.DMA,
            pltpu.SemaphoreType.DMA,
            pltpu.SemaphoreType.DMA,
        ],
    )


def _mlp_body(h_ref, a0_ref, a1_ref, w1_ref, b1_ref, w2_ref, b2_ref, o_ref):
    z = h_ref[...] + a0_ref[...] + a1_ref[...]
    t = jnp.dot(z, w1_ref[...], preferred_element_type=jnp.float32) + b1_ref[...]
    t = jnp.where(t > 0.0, t, jnp.exp(t) - 1.0)
    o_ref[...] = (jnp.dot(t, w2_ref[...], preferred_element_type=jnp.float32)
                  + b2_ref[...])


def _mlp(h, a0, a1, W1, b1, W2, b2, BLK):
    N, D = h.shape
    H = W2.shape[1]
    nb = N // BLK
    row = lambda i: (i, 0)
    full = lambda i: (0, 0)
    return pl.pallas_call(
        _mlp_body,
        grid=(nb,),
        in_specs=[
            pl.BlockSpec((BLK, D), row),
            pl.BlockSpec((BLK, D), row),
            pl.BlockSpec((BLK, D), row),
            pl.BlockSpec((D, H), full),
            pl.BlockSpec((1, H), full),
            pl.BlockSpec((H, H), full),
            pl.BlockSpec((1, H), full),
        ],
        out_specs=pl.BlockSpec((BLK, H), row),
        out_shape=jax.ShapeDtypeStruct((N, H), jnp.float32),
    )(h, a0, a1, W1, b1.reshape(1, H), W2, b2.reshape(1, H))


def _readout_body(nb, BLK, h_ref, brow_ref, bcol_ref, wg_ref, bg_ref,
                  s_ref, m_ref, cnt_ref):
    i = pl.program_id(0)
    H = h_ref.shape[1]
    h = h_ref[...]
    b_row = brow_ref[0]
    b_col = bcol_ref[...]
    w = jax.nn.sigmoid(jnp.dot(h, wg_ref[...],
                               preferred_element_type=jnp.float32)
                       + bg_ref[...])
    wh = w * h
    giT = lax.broadcasted_iota(jnp.int32, (G, BLK), 0)
    onehotT = (giT == b_row).astype(jnp.float32)
    s_c = jnp.dot(onehotT, wh, preferred_element_type=jnp.float32)
    ones = jnp.ones((BLK, H), jnp.float32)
    cnt_c = jnp.dot(onehotT, ones, preferred_element_type=jnp.float32)
    v = h
    s = 1
    while s < BLK:
        vs = jnp.concatenate([v[:s], v[:-s]], axis=0)
        bs_ = jnp.concatenate([jnp.full((s, 1), -1, jnp.int32), b_col[:-s]],
                              axis=0)
        same = b_col == bs_
        v = jnp.where(same, jnp.maximum(v, vs), v)
        s *= 2
    nbrow = jnp.concatenate([b_row[:, 1:], jnp.full((1, 1), -1, jnp.int32)],
                            axis=1)
    is_last = (b_row != nbrow).astype(jnp.float32)
    selT = onehotT * is_last
    m_c = jnp.dot(selT, v, preferred_element_type=jnp.float32)
    m_c = jnp.where(cnt_c > 0.0, m_c, -3e38)

    @pl.when(i == 0)
    def _init():
        s_ref[...] = s_c
        m_ref[...] = m_c
        cnt_ref[...] = cnt_c

    @pl.when(i > 0)
    def _acc():
        s_ref[...] += s_c
        m_ref[...] = jnp.maximum(m_ref[...], m_c)
        cnt_ref[...] += cnt_c

    @pl.when(i == nb - 1)
    def _fin():
        m_ref[...] = jnp.where(cnt_ref[...] > 0.0, m_ref[...], 0.0)


def _readout(h, batch, Wg, bg, BLK):
    N, H = h.shape
    nb = N // BLK
    brow = batch.reshape(nb, 1, BLK)
    bcol = batch.reshape(N, 1)
    full = lambda i: (0, 0)
    s, m = pl.pallas_call(
        functools.partial(_readout_body, nb, BLK),
        grid=(nb,),
        in_specs=[
            pl.BlockSpec((BLK, H), lambda i: (i, 0)),
            pl.BlockSpec((1, 1, BLK), lambda i: (i, 0, 0)),
            pl.BlockSpec((BLK, 1), lambda i: (i, 0)),
            pl.BlockSpec((H, 1), full),
            pl.BlockSpec((1, 1), full),
        ],
        out_specs=[pl.BlockSpec((G, H), full), pl.BlockSpec((G, H), full)],
        out_shape=[jax.ShapeDtypeStruct((G, H), jnp.float32),
                   jax.ShapeDtypeStruct((G, H), jnp.float32)],
        scratch_shapes=[pltpu.VMEM((G, H), jnp.float32)],
        compiler_params=pltpu.CompilerParams(
            dimension_semantics=("arbitrary",)),
    )(h, brow, bcol, Wg, bg.reshape(1, 1))
    return s, m


def kernel(x, edge_index, batch, Ws1, bs1, Ws2, bs2, Wg, bg):
    N, D = x.shape
    L, _, H = Ws1.shape
    E = edge_index.shape[1]
    BLK = 1000

    K = -(-(-(-E // (_NW * _CH))) // _JW) * _JW
    EP = _NW * K * _CH
    NSP = _NS * (-(-(N + 1) // (_NS * 8)) * 8)
    NDUM = NSP - N

    pad = EP - E
    apad = jnp.arange(pad, dtype=jnp.int32)
    src3 = jnp.concatenate([edge_index[0], apad % N]).reshape(_NW, K, _CH)
    dst3 = jnp.concatenate([edge_index[1], N + apad % NDUM]).reshape(_NW, K, _CH)
    zeros = jnp.zeros((NSP // _NS, D), jnp.float32)

    sc_agg = _make_sc_agg(N, D, K, NSP)

    h = x
    for l in range(L):
        agg2 = sc_agg(src3, dst3, h, zeros)
        h = _mlp(h, agg2[0, :N], agg2[1, :N], Ws1[l], bs1[l], Ws2[l], bs2[l],
                 BLK)
    s, m = _readout(h, batch, Wg, bg, BLK)
    return jnp.concatenate([s, m], axis=1)


# speedup vs baseline: 11.1480x; 1.0692x over previous
import functools

import jax
import jax.numpy as jnp
from jax import lax
from jax.experimental import pallas as pl
from jax.experimental.pallas import tpu as pltpu
from jax.experimental.pallas import tpu_sc as plsc

G = 256
_NC, _NS = 2, 16
_NW = _NC * _NS
_CH = 120


_JW = 8


def _sc_agg_body(K, ZCH, src_hbm, dst_hbm, h_hbm, zeros_hbm, out_hbm,
                 srcw0, srcw1, dstw0, dstw1, rows0, rows1, rows2, agg_sp,
                 isem0, isem1, gsem0, gsem1, gsem2):
    cid = lax.axis_index("c")
    sid = lax.axis_index("s")
    pltpu.sync_copy(zeros_hbm, agg_sp.at[pl.ds(sid * ZCH, ZCH)])
    plsc.subcore_barrier()
    w = sid * _NC + cid
    srcw = (srcw0, srcw1)
    dstw = (dstw0, dstw1)
    isem = (isem0, isem1)
    rows = (rows0, rows1, rows2)
    gsem = (gsem0, gsem1, gsem2)
    nwin = K // _JW
    pltpu.sync_copy(src_hbm.at[w, pl.ds(0, _JW)], srcw0)
    pltpu.sync_copy(dst_hbm.at[w, pl.ds(0, _JW)], dstw0)
    gdesc = [None, None, None]
    idesc = [[None, None], [None, None]]
    gdesc[0] = pltpu.async_copy(h_hbm.at[srcw0.at[0]], rows0, gsem0)
    gdesc[1] = pltpu.async_copy(h_hbm.at[srcw0.at[1]], rows1, gsem1)
    for win in range(nwin):
        wb = win & 1
        if win + 1 < nwin:
            nb_ = 1 - wb
            idesc[nb_][0] = pltpu.async_copy(
                src_hbm.at[w, pl.ds((win + 1) * _JW, _JW)], srcw[nb_],
                isem[nb_])
            idesc[nb_][1] = pltpu.async_copy(
                dst_hbm.at[w, pl.ds((win + 1) * _JW, _JW)], dstw[nb_],
                isem[nb_])
        for j2 in range(_JW):
            j = win * _JW + j2
            b = j % 3
            if j2 == _JW - 2 and win + 1 < nwin:
                idesc[1 - wb][0].wait()
                idesc[1 - wb][1].wait()
            if j + 2 < K:
                c = j + 2
                cw = (c // _JW) & 1
                nsrc = srcw[cw].at[c % _JW]
                gdesc[c % 3] = pltpu.async_copy(h_hbm.at[nsrc], rows[c % 3],
                                                gsem[c % 3])
            gdesc[b].wait()
            pltpu.sync_copy(rows[b], agg_sp.at[dstw[wb].at[j2]], add=True)
    plsc.subcore_barrier()
    pltpu.sync_copy(agg_sp.at[pl.ds(sid * ZCH, ZCH)],
                    out_hbm.at[cid, pl.ds(sid * ZCH, ZCH)])


def _make_sc_agg(N, D, K, NSP):
    mesh = plsc.VectorSubcoreMesh(core_axis_name="c", subcore_axis_name="s")
    body = functools.partial(_sc_agg_body, K, NSP // _NS)
    return pl.kernel(
        body,
        out_type=jax.ShapeDtypeStruct((_NC, NSP, D), jnp.float32),
        mesh=mesh,
        scratch_types=[
            pltpu.VMEM((_JW, _CH), jnp.int32),
            pltpu.VMEM((_JW, _CH), jnp.int32),
            pltpu.VMEM((_JW, _CH), jnp.int32),
            pltpu.VMEM((_JW, _CH), jnp.int32),
            pltpu.VMEM((_CH, D), jnp.float32),
            pltpu.VMEM((_CH, D), jnp.float32),
            pltpu.VMEM((_CH, D), jnp.float32),
            pltpu.VMEM_SHARED((NSP, D), jnp.float32),
            pltpu.SemaphoreType.DMA,
            pltpu.SemaphoreType.DMA,
            pltpu.SemaphoreType.DMA,
            pltpu.SemaphoreType.DMA,
            pltpu.SemaphoreType.DMA,
        ],
    )


def _mlp_body(h_ref, a0_ref, a1_ref, w1_ref, b1_ref, w2_ref, b2_ref, o_ref):
    z = h_ref[...] + a0_ref[...] + a1_ref[...]
    t = jnp.dot(z, w1_ref[...], preferred_element_type=jnp.float32) + b1_ref[...]
    t = jnp.where(t > 0.0, t, jnp.exp(t) - 1.0)
    o_ref[...] = (jnp.dot(t, w2_ref[...], preferred_element_type=jnp.float32)
                  + b2_ref[...])


def _mlp(h, a0, a1, W1, b1, W2, b2, BLK):
    N, D = h.shape
    H = W2.shape[1]
    nb = N // BLK
    row = lambda i: (i, 0)
    full = lambda i: (0, 0)
    return pl.pallas_call(
        _mlp_body,
        grid=(nb,),
        in_specs=[
            pl.BlockSpec((BLK, D), row),
            pl.BlockSpec((BLK, D), row),
            pl.BlockSpec((BLK, D), row),
            pl.BlockSpec((D, H), full),
            pl.BlockSpec((1, H), full),
            pl.BlockSpec((H, H), full),
            pl.BlockSpec((1, H), full),
        ],
        out_specs=pl.BlockSpec((BLK, H), row),
        out_shape=jax.ShapeDtypeStruct((N, H), jnp.float32),
    )(h, a0, a1, W1, b1.reshape(1, H), W2, b2.reshape(1, H))


def _readout_body(nb, BLK, h_ref, brow_ref, bcol_ref, wg_ref, bg_ref,
                  s_ref, m_ref, cnt_ref):
    i = pl.program_id(0)
    H = h_ref.shape[1]
    h = h_ref[...]
    b_row = brow_ref[0]
    b_col = bcol_ref[...]
    w = jax.nn.sigmoid(jnp.dot(h, wg_ref[...],
                               preferred_element_type=jnp.float32)
                       + bg_ref[...])
    wh = w * h
    giT = lax.broadcasted_iota(jnp.int32, (G, BLK), 0)
    onehotT = (giT == b_row).astype(jnp.float32)
    s_c = jnp.dot(onehotT, wh, preferred_element_type=jnp.float32)
    ones = jnp.ones((BLK, H), jnp.float32)
    cnt_c = jnp.dot(onehotT, ones, preferred_element_type=jnp.float32)
    v = h
    s = 1
    while s < BLK:
        vs = jnp.concatenate([v[:s], v[:-s]], axis=0)
        bs_ = jnp.concatenate([jnp.full((s, 1), -1, jnp.int32), b_col[:-s]],
                              axis=0)
        same = b_col == bs_
        v = jnp.where(same, jnp.maximum(v, vs), v)
        s *= 2
    nbrow = jnp.concatenate([b_row[:, 1:], jnp.full((1, 1), -1, jnp.int32)],
                            axis=1)
    is_last = (b_row != nbrow).astype(jnp.float32)
    selT = onehotT * is_last
    m_c = jnp.dot(selT, v, preferred_element_type=jnp.float32)
    m_c = jnp.where(cnt_c > 0.0, m_c, -3e38)

    @pl.when(i == 0)
    def _init():
        s_ref[...] = s_c
        m_ref[...] = m_c
        cnt_ref[...] = cnt_c

    @pl.when(i > 0)
    def _acc():
        s_ref[...] += s_c
        m_ref[...] = jnp.maximum(m_ref[...], m_c)
        cnt_ref[...] += cnt_c

    @pl.when(i == nb - 1)
    def _fin():
        m_ref[...] = jnp.where(cnt_ref[...] > 0.0, m_ref[...], 0.0)


def _readout(h, batch, Wg, bg, BLK):
    N, H = h.shape
    nb = N // BLK
    brow = batch.reshape(nb, 1, BLK)
    bcol = batch.reshape(N, 1)
    full = lambda i: (0, 0)
    s, m = pl.pallas_call(
        functools.partial(_readout_body, nb, BLK),
        grid=(nb,),
        in_specs=[
            pl.BlockSpec((BLK, H), lambda i: (i, 0)),
            pl.BlockSpec((1, 1, BLK), lambda i: (i, 0, 0)),
            pl.BlockSpec((BLK, 1), lambda i: (i, 0)),
            pl.BlockSpec((H, 1), full),
            pl.BlockSpec((1, 1), full),
        ],
        out_specs=[pl.BlockSpec((G, H), full), pl.BlockSpec((G, H), full)],
        out_shape=[jax.ShapeDtypeStruct((G, H), jnp.float32),
                   jax.ShapeDtypeStruct((G, H), jnp.float32)],
        scratch_shapes=[pltpu.VMEM((G, H), jnp.float32)],
        compiler_params=pltpu.CompilerParams(
            dimension_semantics=("arbitrary",)),
    )(h, brow, bcol, Wg, bg.reshape(1, 1))
    return s, m


def kernel(x, edge_index, batch, Ws1, bs1, Ws2, bs2, Wg, bg):
    N, D = x.shape
    L, _, H = Ws1.shape
    E = edge_index.shape[1]
    BLK = 1000

    K = -(-(-(-E // (_NW * _CH))) // _JW) * _JW
    EP = _NW * K * _CH
    NSP = _NS * (-(-(N + 1) // (_NS * 8)) * 8)
    NDUM = NSP - N

    pad = EP - E
    apad = jnp.arange(pad, dtype=jnp.int32)
    src3 = jnp.concatenate([edge_index[0], apad % N]).reshape(_NW, K, _CH)
    dst3 = jnp.concatenate([edge_index[1], N + apad % NDUM]).reshape(_NW, K, _CH)
    zeros = jnp.zeros((NSP // _NS, D), jnp.float32)

    sc_agg = _make_sc_agg(N, D, K, NSP)

    h = x
    for l in range(L):
        agg2 = sc_agg(src3, dst3, h, zeros)
        h = _mlp(h, agg2[0, :N], agg2[1, :N], Ws1[l], bs1[l], Ws2[l], bs2[l],
                 BLK)
    s, m = _readout(h, batch, Wg, bg, BLK)
    return jnp.concatenate([s, m], axis=1)


# speedup vs baseline: 12.0708x; 1.0828x over previous
import functools

import jax
import jax.numpy as jnp
from jax import lax
from jax.experimental import pallas as pl
from jax.experimental.pallas import tpu as pltpu
from jax.experimental.pallas import tpu_sc as plsc

G = 256
_NC, _NS = 2, 16
_NW = _NC * _NS
_CH = 120


_JW = 8


def _sc_agg_body(K, ZCH, src_hbm, dst_hbm, h_hbm, zeros_hbm, out_hbm,
                 srcw0, srcw1, dstw0, dstw1, rows0, rows1, rows2, agg_sp,
                 isem0, isem1, gsem0, gsem1, gsem2):
    cid = lax.axis_index("c")
    sid = lax.axis_index("s")
    pltpu.sync_copy(zeros_hbm, agg_sp.at[pl.ds(sid * ZCH, ZCH)])
    plsc.subcore_barrier()
    w = sid * _NC + cid
    srcw = (srcw0, srcw1)
    dstw = (dstw0, dstw1)
    isem = (isem0, isem1)
    rows = (rows0, rows1, rows2)
    gsem = (gsem0, gsem1, gsem2)
    nwin = K // _JW
    pltpu.sync_copy(src_hbm.at[w, pl.ds(0, _JW)], srcw0)
    pltpu.sync_copy(dst_hbm.at[w, pl.ds(0, _JW)], dstw0)
    gdesc = [None, None, None]
    idesc = [[None, None], [None, None]]
    gdesc[0] = pltpu.async_copy(h_hbm.at[srcw0.at[0]], rows0, gsem0)
    gdesc[1] = pltpu.async_copy(h_hbm.at[srcw0.at[1]], rows1, gsem1)
    for win in range(nwin):
        wb = win & 1
        if win + 1 < nwin:
            nb_ = 1 - wb
            idesc[nb_][0] = pltpu.async_copy(
                src_hbm.at[w, pl.ds((win + 1) * _JW, _JW)], srcw[nb_],
                isem[nb_])
            idesc[nb_][1] = pltpu.async_copy(
                dst_hbm.at[w, pl.ds((win + 1) * _JW, _JW)], dstw[nb_],
                isem[nb_])
        for j2 in range(_JW):
            j = win * _JW + j2
            b = j % 3
            if j2 == _JW - 2 and win + 1 < nwin:
                idesc[1 - wb][0].wait()
                idesc[1 - wb][1].wait()
            if j + 2 < K:
                c = j + 2
                cw = (c // _JW) & 1
                nsrc = srcw[cw].at[c % _JW]
                gdesc[c % 3] = pltpu.async_copy(h_hbm.at[nsrc], rows[c % 3],
                                                gsem[c % 3])
            gdesc[b].wait()
            pltpu.sync_copy(rows[b], agg_sp.at[dstw[wb].at[j2]], add=True)
    plsc.subcore_barrier()
    pltpu.sync_copy(agg_sp.at[pl.ds(sid * ZCH, ZCH)],
                    out_hbm.at[cid, pl.ds(sid * ZCH, ZCH)])


def _make_sc_agg(N, D, K, NSP):
    mesh = plsc.VectorSubcoreMesh(core_axis_name="c", subcore_axis_name="s")
    body = functools.partial(_sc_agg_body, K, NSP // _NS)
    return pl.kernel(
        body,
        out_type=jax.ShapeDtypeStruct((_NC, NSP, D), jnp.float32),
        mesh=mesh,
        scratch_types=[
            pltpu.VMEM((_JW, _CH), jnp.int32),
            pltpu.VMEM((_JW, _CH), jnp.int32),
            pltpu.VMEM((_JW, _CH), jnp.int32),
            pltpu.VMEM((_JW, _CH), jnp.int32),
            pltpu.VMEM((_CH, D), jnp.float32),
            pltpu.VMEM((_CH, D), jnp.float32),
            pltpu.VMEM((_CH, D), jnp.float32),
            pltpu.VMEM_SHARED((NSP, D), jnp.float32),
            pltpu.SemaphoreType.DMA,
            pltpu.SemaphoreType.DMA,
            pltpu.SemaphoreType.DMA,
            pltpu.SemaphoreType.DMA,
            pltpu.SemaphoreType.DMA,
        ],
    )


def _mlp_block(h_ref, a0_ref, a1_ref, w1_ref, b1_ref, w2_ref, b2_ref):
    z = h_ref[...] + a0_ref[0] + a1_ref[0]
    t = jnp.dot(z, w1_ref[...], preferred_element_type=jnp.float32) + b1_ref[...]
    t = jnp.where(t > 0.0, t, jnp.exp(t) - 1.0)
    return (jnp.dot(t, w2_ref[...], preferred_element_type=jnp.float32)
            + b2_ref[...])


def _mlp_body(h_ref, a0_ref, a1_ref, w1_ref, b1_ref, w2_ref, b2_ref, o_ref):
    o_ref[...] = _mlp_block(h_ref, a0_ref, a1_ref, w1_ref, b1_ref, w2_ref,
                            b2_ref)


def _mlp_specs(BLK, D, H):
    row = lambda i: (i, 0)
    full = lambda i: (0, 0)
    return [
        pl.BlockSpec((BLK, D), row),
        pl.BlockSpec((1, BLK, D), lambda i: (0, i, 0)),
        pl.BlockSpec((1, BLK, D), lambda i: (1, i, 0)),
        pl.BlockSpec((D, H), full),
        pl.BlockSpec((1, H), full),
        pl.BlockSpec((H, H), full),
        pl.BlockSpec((1, H), full),
    ]


def _mlp(h, agg2, W1, b1, W2, b2, BLK):
    N, D = h.shape
    H = W2.shape[1]
    nb = N // BLK
    return pl.pallas_call(
        _mlp_body,
        grid=(nb,),
        in_specs=_mlp_specs(BLK, D, H),
        out_specs=pl.BlockSpec((BLK, H), lambda i: (i, 0)),
        out_shape=jax.ShapeDtypeStruct((N, H), jnp.float32),
    )(h, agg2, agg2, W1, b1.reshape(1, H), W2, b2.reshape(1, H))


def _readout_body(nb, BLK, h_ref, a0_ref, a1_ref, w1_ref, b1_ref, w2_ref,
                  b2_ref, brow_ref, bcol_ref, wg_ref, bg_ref,
                  s_ref, m_ref, cnt_ref):
    i = pl.program_id(0)
    H = w2_ref.shape[1]
    h = _mlp_block(h_ref, a0_ref, a1_ref, w1_ref, b1_ref, w2_ref, b2_ref)
    b_row = brow_ref[0]
    b_col = bcol_ref[...]
    w = jax.nn.sigmoid(jnp.dot(h, wg_ref[...],
                               preferred_element_type=jnp.float32)
                       + bg_ref[...])
    wh = w * h
    giT = lax.broadcasted_iota(jnp.int32, (G, BLK), 0)
    onehotT = (giT == b_row).astype(jnp.float32)
    s_c = jnp.dot(onehotT, wh, preferred_element_type=jnp.float32)
    ones = jnp.ones((BLK, H), jnp.float32)
    cnt_c = jnp.dot(onehotT, ones, preferred_element_type=jnp.float32)
    v = h
    s = 1
    while s < BLK:
        vs = jnp.concatenate([v[:s], v[:-s]], axis=0)
        bs_ = jnp.concatenate([jnp.full((s, 1), -1, jnp.int32), b_col[:-s]],
                              axis=0)
        same = b_col == bs_
        v = jnp.where(same, jnp.maximum(v, vs), v)
        s *= 2
    nbrow = jnp.concatenate([b_row[:, 1:], jnp.full((1, 1), -1, jnp.int32)],
                            axis=1)
    is_last = (b_row != nbrow).astype(jnp.float32)
    selT = onehotT * is_last
    m_c = jnp.dot(selT, v, preferred_element_type=jnp.float32)
    m_c = jnp.where(cnt_c > 0.0, m_c, -3e38)

    @pl.when(i == 0)
    def _init():
        s_ref[...] = s_c
        m_ref[...] = m_c
        cnt_ref[...] = cnt_c

    @pl.when(i > 0)
    def _acc():
        s_ref[...] += s_c
        m_ref[...] = jnp.maximum(m_ref[...], m_c)
        cnt_ref[...] += cnt_c

    @pl.when(i == nb - 1)
    def _fin():
        m_ref[...] = jnp.where(cnt_ref[...] > 0.0, m_ref[...], 0.0)


def _readout(h, agg2, W1, b1, W2, b2, batch, Wg, bg, BLK):
    N, D = h.shape
    H = W2.shape[1]
    nb = N // BLK
    brow = batch.reshape(nb, 1, BLK)
    bcol = batch.reshape(N, 1)
    full = lambda i: (0, 0)
    s, m = pl.pallas_call(
        functools.partial(_readout_body, nb, BLK),
        grid=(nb,),
        in_specs=_mlp_specs(BLK, D, H) + [
            pl.BlockSpec((1, 1, BLK), lambda i: (i, 0, 0)),
            pl.BlockSpec((BLK, 1), lambda i: (i, 0)),
            pl.BlockSpec((H, 1), full),
            pl.BlockSpec((1, 1), full),
        ],
        out_specs=[pl.BlockSpec((G, H), full), pl.BlockSpec((G, H), full)],
        out_shape=[jax.ShapeDtypeStruct((G, H), jnp.float32),
                   jax.ShapeDtypeStruct((G, H), jnp.float32)],
        scratch_shapes=[pltpu.VMEM((G, H), jnp.float32)],
        compiler_params=pltpu.CompilerParams(
            dimension_semantics=("arbitrary",)),
    )(h, agg2, agg2, W1, b1.reshape(1, H), W2, b2.reshape(1, H),
      brow, bcol, Wg, bg.reshape(1, 1))
    return s, m


def kernel(x, edge_index, batch, Ws1, bs1, Ws2, bs2, Wg, bg):
    N, D = x.shape
    L, _, H = Ws1.shape
    E = edge_index.shape[1]
    BLK = 1000

    K = -(-(-(-E // (_NW * _CH))) // _JW) * _JW
    EP = _NW * K * _CH
    NSP = _NS * (-(-(N + 1) // (_NS * 8)) * 8)
    NDUM = NSP - N

    pad = EP - E
    apad = jnp.arange(pad, dtype=jnp.int32)
    src3 = jnp.concatenate([edge_index[0], apad % N]).reshape(_NW, K, _CH)
    dst3 = jnp.concatenate([edge_index[1], N + apad % NDUM]).reshape(_NW, K, _CH)
    zeros = jnp.zeros((NSP // _NS, D), jnp.float32)

    sc_agg = _make_sc_agg(N, D, K, NSP)

    h = x
    for l in range(L - 1):
        agg2 = sc_agg(src3, dst3, h, zeros)
        h = _mlp(h, agg2, Ws1[l], bs1[l], Ws2[l], bs2[l], BLK)
    agg2 = sc_agg(src3, dst3, h, zeros)
    s, m = _readout(h, agg2, Ws1[L - 1], bs1[L - 1], Ws2[L - 1], bs2[L - 1],
                    batch, Wg, bg, BLK)
    return jnp.concatenate([s, m], axis=1)
